# Initial kernel scaffold; baseline (speedup 1.0000x reference)
import jax
import jax.numpy as jnp
from jax.experimental import pallas as pl


def kernel(node_pos, state_in, time_i, conditions, spatial_inform, edges, params):
    raise NotImplementedError("write your pallas kernel here")


# speedup vs baseline: 1.5480x; 1.5480x over previous
import functools

import jax
import jax.numpy as jnp
from jax import lax
from jax.experimental import pallas as pl
from jax.experimental.pallas import tpu as pltpu
from jax.experimental.pallas import tpu_sc as plsc

ENC = 128
NB = 4
NF = 2
S = 3
N_NODES = 10000
NE = 160000
NW = 32
CHUNK = 128
NE_PAD = 163840
K_CH = NE_PAD // NW // CHUNK
R_PAD = 10240
NT = 1000
ET = 1024

F32 = jnp.float32


def _make_dual_gather(cols):
    mesh = plsc.VectorSubcoreMesh(core_axis_name="c", subcore_axis_name="s")

    @functools.partial(
        pl.kernel,
        out_type=[jax.ShapeDtypeStruct((NE_PAD, cols), F32),
                  jax.ShapeDtypeStruct((NE_PAD, cols), F32)],
        mesh=mesh,
        compiler_params=pltpu.CompilerParams(use_tc_tiling_on_sc=(cols % 128 == 0)),
        scratch_types=[
            pltpu.VMEM((K_CH, CHUNK), jnp.int32),
            pltpu.VMEM((K_CH, CHUNK), jnp.int32),
            pltpu.VMEM((CHUNK, cols), F32),
            pltpu.VMEM((CHUNK, cols), F32),
            pltpu.SemaphoreType.DMA,
            pltpu.SemaphoreType.DMA,
        ],
    )
    def gather2(tab_a, idx_a, tab_b, idx_b, out_a, out_b,
                idx_va, idx_vb, buf_a, buf_b, sem_a, sem_b):
        cid = lax.axis_index("c")
        sid = lax.axis_index("s")
        wid = sid * 2 + cid
        base = wid * K_CH
        pltpu.sync_copy(idx_a.at[pl.ds(base, K_CH)], idx_va)
        pltpu.sync_copy(idx_b.at[pl.ds(base, K_CH)], idx_vb)

        def body(j, carry):
            ca = pltpu.async_copy(tab_a.at[idx_va.at[j]], buf_a, sem_a)
            cb = pltpu.async_copy(tab_b.at[idx_vb.at[j]], buf_b, sem_b)
            ca.wait()
            cb.wait()
            row0 = (base + j) * CHUNK
            pltpu.sync_copy(buf_a, out_a.at[pl.ds(row0, CHUNK)])
            pltpu.sync_copy(buf_b, out_b.at[pl.ds(row0, CHUNK)])
            return carry

        lax.fori_loop(0, K_CH, body, 0)

    return gather2


def _make_scatter_add():
    mesh = plsc.VectorSubcoreMesh(core_axis_name="c", subcore_axis_name="s")
    stripe = R_PAD // 16

    @functools.partial(
        pl.kernel,
        out_type=jax.ShapeDtypeStruct((2, R_PAD, ENC), F32),
        mesh=mesh,
        scratch_types=[
            pltpu.VMEM((K_CH, CHUNK), jnp.int32),
            pltpu.VMEM((CHUNK, ENC), F32),
            pltpu.VMEM_SHARED((R_PAD, ENC), F32),
        ],
    )
    def scatter(m_hbm, idx_hbm, zeros_hbm, out, idx_v, buf, shared):
        cid = lax.axis_index("c")
        sid = lax.axis_index("s")
        wid = sid * 2 + cid
        pltpu.sync_copy(zeros_hbm.at[pl.ds(sid * stripe, stripe)],
                        shared.at[pl.ds(sid * stripe, stripe)])
        plsc.subcore_barrier()
        base = wid * K_CH
        pltpu.sync_copy(idx_hbm.at[pl.ds(base, K_CH)], idx_v)

        def body(j, carry):
            pltpu.sync_copy(m_hbm.at[pl.ds((base + j) * CHUNK, CHUNK)], buf)
            pltpu.sync_copy(buf, shared.at[idx_v.at[j]], add=True)
            return carry

        lax.fori_loop(0, K_CH, body, 0)
        plsc.subcore_barrier()
        pltpu.sync_copy(shared.at[pl.ds(sid * stripe, stripe)],
                        out.at[cid].at[pl.ds(sid * stripe, stripe)])

    return scatter


def _dot(a, b):
    return jnp.dot(a, b, preferred_element_type=F32)


def _glob_body(t_ref, c_ref, sp_ref,
               wt1, bt1, wt2, bt2, wc1, bc1, wc2, bc2, ws1, bs1, ws2, bs2,
               fw1, fb1, fw2, fb2, para_ref):
    te = _dot(jax.nn.silu(_dot(t_ref[...], wt1[...]) + bt1[...]), wt2[...]) + bt2[...]
    ce = _dot(jax.nn.silu(_dot(c_ref[...], wc1[...]) + bc1[...]), wc2[...]) + bc2[...]
    se = _dot(jax.nn.silu(_dot(sp_ref[...], ws1[...]) + bs1[...]), ws2[...]) + bs2[...]
    h = jnp.concatenate([ce, te, se], axis=-1)
    hh = jax.nn.silu(_dot(h, fw1[...]) + fb1[...])
    para_ref[...] = _dot(hh, fw2[...]) + fb2[...]


def _edge_enc_body(spos_ref, rpos_ref, w1, b1, w2, b2, e_ref):
    d = rpos_ref[...] - spos_ref[...]
    nsq = jnp.sum(d * d, axis=1, keepdims=True) + 1e-8
    nrm = jnp.sqrt(nsq)
    d3 = d[:, :3]
    feat = jnp.concatenate(
        [d3, -d3, nrm, jnp.zeros((d.shape[0], 9), F32)], axis=-1)
    h = jax.nn.silu(_dot(feat, w1[...]) + b1[...])
    e_ref[...] = _dot(h, w2[...]) + b2[...]


def _node_enc_body(x_ref, w1, b1, w2, b2, para_ref, v_ref):
    h = jax.nn.silu(_dot(x_ref[...], w1[...]) + b1[...])
    v0 = _dot(h, w2[...]) + b2[...]
    gamma = para_ref[0:1, :ENC]
    beta = para_ref[0:1, ENC:]
    v_ref[...] = v0 * gamma + beta


def _edge_mlp_body(s_ref, r_ref, ef_ref, we1, be1, we2, be2, m_ref, efn_ref):
    ef = ef_ref[...]
    x = jnp.concatenate([s_ref[...], r_ref[...], ef], axis=-1)
    pre = _dot(x, we1[...]) + be1[...]
    m = _dot(jax.nn.silu(pre), we2[...]) + be2[...]
    m_ref[...] = m
    efn_ref[...] = ef + m


def _node_mlp_body(v_ref, a0_ref, a1_ref, wn1, bn1, wn2, bn2, vn_ref):
    v = v_ref[...]
    agg = a0_ref[0] + a1_ref[0]
    x = jnp.concatenate([v, agg], axis=-1)
    h = jax.nn.silu(_dot(x, wn1[...]) + bn1[...])
    vn_ref[...] = v + _dot(h, wn2[...]) + bn2[...]


def _decoder_body(x_ref, spread, v1_ref, v2_ref, v3_ref, v4_ref,
                  aw1, ab1, aw2, ab2, dw1, db1, dw2, db2, dw3, db3,
                  out_ref):
    x = x_ref[...]
    sp = spread[...]
    y = (x[:, 0:1] * sp[0:1, :] + x[:, 1:2] * sp[1:2, :]
         + x[:, 2:3] * sp[2:3, :])
    lane = lax.broadcasted_iota(jnp.int32, y.shape, 1)
    pe = jnp.where(lane < 18,
                   jnp.where(lane % 6 < 3, jnp.cos(y), jnp.sin(y)),
                   jnp.where(lane < 21, y, 0.0))
    a = _dot(jax.nn.silu(_dot(pe, aw1[...]) + ab1[...]), aw2[...]) + ab2[...]
    a = jnp.clip(a, -30.0, 30.0)
    valid = lane < 4
    al = jnp.where(valid, a, -1e30)
    mx = jnp.max(al, axis=1, keepdims=True)
    e = jnp.where(valid, jnp.exp(al - mx), 0.0)
    w = e / jnp.sum(e, axis=1, keepdims=True)
    vagg = (w[:, 0:1] * v1_ref[...] + w[:, 1:2] * v2_ref[...]
            + w[:, 2:3] * v3_ref[...] + w[:, 3:4] * v4_ref[...])
    dd = jnp.concatenate([vagg, pe], axis=-1)
    h = jax.nn.silu(_dot(dd, dw1[...]) + db1[...])
    h = jax.nn.silu(_dot(h, dw2[...]) + db2[...])
    o = _dot(h, dw3[...])
    res = o[:, 0:1] + db3[...][0:1, 0:1] + x[:, 3:4]
    out_ref[...] = jnp.broadcast_to(res, out_ref.shape)


def _full(shape):
    return pl.BlockSpec(shape, lambda i: (0,) * len(shape))


def _glob_call(t, c, sp, *ws):
    specs = [_full(x.shape) for x in (t, c, sp)] + [_full(w.shape) for w in ws]
    return pl.pallas_call(
        _glob_body,
        grid=(1,),
        in_specs=specs,
        out_specs=_full((8, 2 * ENC)),
        out_shape=jax.ShapeDtypeStruct((8, 2 * ENC), F32),
    )(t, c, sp, *ws)


def _edge_enc_call(spos, rpos, w1, b1, w2, b2):
    g = NE_PAD // ET
    espec = pl.BlockSpec((ET, 16), lambda i: (i, 0))
    return pl.pallas_call(
        _edge_enc_body,
        grid=(g,),
        in_specs=[espec, espec] + [_full(w.shape) for w in (w1, b1, w2, b2)],
        out_specs=pl.BlockSpec((ET, ENC), lambda i: (i, 0)),
        out_shape=jax.ShapeDtypeStruct((NE_PAD, ENC), F32),
    )(spos, rpos, w1, b1, w2, b2)


def _node_enc_call(x, w1, b1, w2, b2, para):
    g = N_NODES // NT
    nspec = pl.BlockSpec((NT, ENC), lambda i: (i, 0))
    return pl.pallas_call(
        _node_enc_body,
        grid=(g,),
        in_specs=[pl.BlockSpec((NT, 8), lambda i: (i, 0))]
        + [_full(w.shape) for w in (w1, b1, w2, b2, para)],
        out_specs=nspec,
        out_shape=jax.ShapeDtypeStruct((N_NODES, ENC), F32),
    )(x, w1, b1, w2, b2, para)


def _edge_mlp_call(s, r, ef, we1, be1, we2, be2):
    g = NE_PAD // ET
    espec = pl.BlockSpec((ET, ENC), lambda i: (i, 0))
    return pl.pallas_call(
        _edge_mlp_body,
        grid=(g,),
        in_specs=[espec, espec, espec]
        + [_full(w.shape) for w in (we1, be1, we2, be2)],
        out_specs=[espec, espec],
        out_shape=[jax.ShapeDtypeStruct((NE_PAD, ENC), F32)] * 2,
    )(s, r, ef, we1, be1, we2, be2)


def _node_mlp_call(v, aggs, wn1, bn1, wn2, bn2):
    g = N_NODES // NT
    nspec = pl.BlockSpec((NT, ENC), lambda i: (i, 0))
    a0spec = pl.BlockSpec((1, NT, ENC), lambda i: (0, i, 0))
    a1spec = pl.BlockSpec((1, NT, ENC), lambda i: (1, i, 0))
    return pl.pallas_call(
        _node_mlp_body,
        grid=(g,),
        in_specs=[nspec, a0spec, a1spec]
        + [_full(w.shape) for w in (wn1, bn1, wn2, bn2)],
        out_specs=nspec,
        out_shape=jax.ShapeDtypeStruct((N_NODES, ENC), F32),
    )(v, aggs, aggs, wn1, bn1, wn2, bn2)


def _decoder_call(x, spread, v1, v2, v3, v4, *ws):
    g = N_NODES // NT
    nspec = pl.BlockSpec((NT, ENC), lambda i: (i, 0))
    xspec = pl.BlockSpec((NT, 8), lambda i: (i, 0))
    return pl.pallas_call(
        _decoder_body,
        grid=(g,),
        in_specs=[xspec, _full(spread.shape), nspec, nspec, nspec, nspec]
        + [_full(w.shape) for w in ws],
        out_specs=xspec,
        out_shape=jax.ShapeDtypeStruct((N_NODES, 8), F32),
    )(x, spread, v1, v2, v3, v4, *ws)


def _row128(vec, width):
    out = jnp.zeros((8, 128), F32)
    return out.at[0, :width].set(vec)


def _padw(w, rows, cols):
    return jnp.zeros((rows, cols), F32).at[:w.shape[0], :w.shape[1]].set(w)


def kernel(node_pos, state_in, time_i, conditions, spatial_inform, edges, params):
    p = params
    pos = node_pos[0]
    state = state_in[0]

    send = edges[0, :, 0].astype(jnp.int32)
    recv = edges[0, :, 1].astype(jnp.int32)
    pad = NE_PAD - NE
    send_g = jnp.concatenate([send, jnp.zeros((pad,), jnp.int32)]).reshape(-1, CHUNK)
    recv_g = jnp.concatenate([recv, jnp.zeros((pad,), jnp.int32)]).reshape(-1, CHUNK)
    recv_s = jnp.concatenate(
        [recv, jnp.full((pad,), N_NODES, jnp.int32)]).reshape(-1, CHUNK)
    zeros_acc = jnp.zeros((R_PAD, ENC), F32)

    gather16 = _make_dual_gather(16)
    gather128 = _make_dual_gather(ENC)
    scatter_add = _make_scatter_add()

    para = _glob_call(
        _row128(time_i[0], 11), _row128(conditions[0], 32),
        _row128(spatial_inform[0], 10),
        _padw(p['ft_W1'], 128, 128), p['ft_b1'][None], p['ft_W2'], p['ft_b2'][None],
        _padw(p['fc_W1'], 128, 128), p['fc_b1'][None], p['fc_W2'], p['fc_b2'][None],
        _padw(p['fs_W1'], 128, 128), p['fs_b1'][None], p['fs_W2'], p['fs_b2'][None],
        p['fu_W1'], p['fu_b1'][None], p['fu_W2'], p['fu_b2'][None])

    pos16 = jnp.zeros((N_NODES, 16), F32).at[:, :S].set(pos)
    spos, rpos = gather16(pos16, send_g, pos16, recv_g)
    E = _edge_enc_call(
        spos, rpos, _padw(p['fe_W1'], 16, 128), p['fe_b1'][None],
        p['fe_W2'], p['fe_b2'][None])

    x_nodes = jnp.zeros((N_NODES, 8), F32).at[:, :S].set(pos)
    spread = jnp.zeros((8, 128), F32)
    freq = [float(jnp.pi), float(2 * jnp.pi), float(4 * jnp.pi)]
    for j in range(3):
        for k in range(3):
            spread = spread.at[j, 6 * j + k].set(freq[k])
            spread = spread.at[j, 6 * j + 3 + k].set(freq[k])
        spread = spread.at[j, 18 + j].set(1.0)

    outs = []
    for f in range(NF):
        xf = x_nodes.at[:, 3].set(state[:, f])
        v = _node_enc_call(
            xf, _padw(p[f'fvf{f}_W1'], 8, 128), p[f'fvf{f}_b1'][None],
            p[f'fvf{f}_W2'], p[f'fvf{f}_b2'][None], para)
        ef = E
        vs = []
        for b in range(NB):
            sg, rg = gather128(v, send_g, v, recv_g)
            m, ef = _edge_mlp_call(sg, rg, ef, p[f'blk{b}_We1'],
                                   p[f'blk{b}_be1'][None],
                                   p[f'blk{b}_We2'], p[f'blk{b}_be2'][None])
            aggs = scatter_add(m, recv_s, zeros_acc)
            v = _node_mlp_call(v, aggs, p[f'blk{b}_Wn1'], p[f'blk{b}_bn1'][None],
                               p[f'blk{b}_Wn2'], p[f'blk{b}_bn2'][None])
            vs.append(v)

        dw1 = jnp.concatenate(
            [p[f'dec{f}_dW1'][:ENC], _padw(p[f'dec{f}_dW1'][ENC:], 128, 128)],
            axis=0)
        out_f = _decoder_call(
            xf, spread, *vs,
            _padw(p[f'dec{f}_aW1'], 128, 128), p[f'dec{f}_ab1'][None],
            _padw(p[f'dec{f}_aW2'], 128, 128), _padw(p[f'dec{f}_ab2'][None], 1, 128),
            dw1, p[f'dec{f}_db1'][None], p[f'dec{f}_dW2'], p[f'dec{f}_db2'][None],
            _padw(p[f'dec{f}_dW3'], 128, 8), _padw(p[f'dec{f}_db3'][None], 1, 8))
        outs.append(out_f[:, 0])

    return jnp.stack(outs, axis=-1)[None]


# speedup vs baseline: 1.5819x; 1.0219x over previous
import functools

import jax
import jax.numpy as jnp
from jax import lax
from jax.experimental import pallas as pl
from jax.experimental.pallas import tpu as pltpu
from jax.experimental.pallas import tpu_sc as plsc

ENC = 128
NB = 4
NF = 2
S = 3
N_NODES = 10000
NE = 160000
NW = 32
CHUNK = 128
NE_PAD = 163840
K_CH = NE_PAD // NW // CHUNK
R_PAD = 10240
NT = 1000
ET = 1024

F32 = jnp.float32


def _make_dual_gather(cols):
    mesh = plsc.VectorSubcoreMesh(core_axis_name="c", subcore_axis_name="s")

    @functools.partial(
        pl.kernel,
        out_type=[jax.ShapeDtypeStruct((NE_PAD, cols), F32),
                  jax.ShapeDtypeStruct((NE_PAD, cols), F32)],
        mesh=mesh,
        compiler_params=pltpu.CompilerParams(use_tc_tiling_on_sc=(cols % 128 == 0)),
        scratch_types=[
            pltpu.VMEM((K_CH, CHUNK), jnp.int32),
            pltpu.VMEM((K_CH, CHUNK), jnp.int32),
            pltpu.VMEM((CHUNK, cols), F32),
            pltpu.VMEM((CHUNK, cols), F32),
            pltpu.VMEM((CHUNK, cols), F32),
            pltpu.VMEM((CHUNK, cols), F32),
            pltpu.SemaphoreType.DMA,
            pltpu.SemaphoreType.DMA,
            pltpu.SemaphoreType.DMA,
            pltpu.SemaphoreType.DMA,
        ],
    )
    def gather2(tab_a, idx_a, tab_b, idx_b, out_a, out_b,
                idx_va, idx_vb, buf_a0, buf_b0, buf_a1, buf_b1,
                sem_a0, sem_b0, sem_a1, sem_b1):
        cid = lax.axis_index("c")
        sid = lax.axis_index("s")
        wid = sid * 2 + cid
        base = wid * K_CH
        pltpu.sync_copy(idx_a.at[pl.ds(base, K_CH)], idx_va)
        pltpu.sync_copy(idx_b.at[pl.ds(base, K_CH)], idx_vb)

        @pl.loop(0, K_CH, step=2)
        def body(j):
            ca0 = pltpu.async_copy(tab_a.at[idx_va.at[j]], buf_a0, sem_a0)
            cb0 = pltpu.async_copy(tab_b.at[idx_vb.at[j]], buf_b0, sem_b0)
            ca1 = pltpu.async_copy(tab_a.at[idx_va.at[j + 1]], buf_a1, sem_a1)
            cb1 = pltpu.async_copy(tab_b.at[idx_vb.at[j + 1]], buf_b1, sem_b1)
            row0 = (base + j) * CHUNK
            ca0.wait()
            cb0.wait()
            pltpu.sync_copy(buf_a0, out_a.at[pl.ds(row0, CHUNK)])
            pltpu.sync_copy(buf_b0, out_b.at[pl.ds(row0, CHUNK)])
            ca1.wait()
            cb1.wait()
            pltpu.sync_copy(buf_a1, out_a.at[pl.ds(row0 + CHUNK, CHUNK)])
            pltpu.sync_copy(buf_b1, out_b.at[pl.ds(row0 + CHUNK, CHUNK)])

    return gather2


def _make_scatter_add():
    mesh = plsc.VectorSubcoreMesh(core_axis_name="c", subcore_axis_name="s")
    stripe = R_PAD // 16

    @functools.partial(
        pl.kernel,
        out_type=jax.ShapeDtypeStruct((2, R_PAD, ENC), F32),
        mesh=mesh,
        scratch_types=[
            pltpu.VMEM((K_CH, CHUNK), jnp.int32),
            pltpu.VMEM((CHUNK, ENC), F32),
            pltpu.VMEM((CHUNK, ENC), F32),
            pltpu.VMEM_SHARED((R_PAD, ENC), F32),
            pltpu.SemaphoreType.DMA,
            pltpu.SemaphoreType.DMA,
        ],
    )
    def scatter(m_hbm, idx_hbm, zeros_hbm, out, idx_v, buf0, buf1, shared,
                sem0, sem1):
        cid = lax.axis_index("c")
        sid = lax.axis_index("s")
        wid = sid * 2 + cid
        pltpu.sync_copy(zeros_hbm.at[pl.ds(sid * stripe, stripe)],
                        shared.at[pl.ds(sid * stripe, stripe)])
        plsc.subcore_barrier()
        base = wid * K_CH
        pltpu.sync_copy(idx_hbm.at[pl.ds(base, K_CH)], idx_v)

        @pl.loop(0, K_CH, step=2)
        def body(j):
            c0 = pltpu.async_copy(m_hbm.at[pl.ds((base + j) * CHUNK, CHUNK)],
                                  buf0, sem0)
            c1 = pltpu.async_copy(m_hbm.at[pl.ds((base + j + 1) * CHUNK, CHUNK)],
                                  buf1, sem1)
            c0.wait()
            pltpu.sync_copy(buf0, shared.at[idx_v.at[j]], add=True)
            c1.wait()
            pltpu.sync_copy(buf1, shared.at[idx_v.at[j + 1]], add=True)
        plsc.subcore_barrier()
        pltpu.sync_copy(shared.at[pl.ds(sid * stripe, stripe)],
                        out.at[cid].at[pl.ds(sid * stripe, stripe)])

    return scatter


def _dot(a, b):
    return jnp.dot(a, b, preferred_element_type=F32)


def _glob_body(t_ref, c_ref, sp_ref,
               wt1, bt1, wt2, bt2, wc1, bc1, wc2, bc2, ws1, bs1, ws2, bs2,
               fw1, fb1, fw2, fb2, para_ref):
    te = _dot(jax.nn.silu(_dot(t_ref[...], wt1[...]) + bt1[...]), wt2[...]) + bt2[...]
    ce = _dot(jax.nn.silu(_dot(c_ref[...], wc1[...]) + bc1[...]), wc2[...]) + bc2[...]
    se = _dot(jax.nn.silu(_dot(sp_ref[...], ws1[...]) + bs1[...]), ws2[...]) + bs2[...]
    h = jnp.concatenate([ce, te, se], axis=-1)
    hh = jax.nn.silu(_dot(h, fw1[...]) + fb1[...])
    para_ref[...] = _dot(hh, fw2[...]) + fb2[...]


def _edge_enc_body(spos_ref, rpos_ref, w1, b1, w2, b2, e_ref):
    d = rpos_ref[...] - spos_ref[...]
    nsq = jnp.sum(d * d, axis=1, keepdims=True) + 1e-8
    nrm = jnp.sqrt(nsq)
    d3 = d[:, :3]
    feat = jnp.concatenate(
        [d3, -d3, nrm, jnp.zeros((d.shape[0], 9), F32)], axis=-1)
    h = jax.nn.silu(_dot(feat, w1[...]) + b1[...])
    e_ref[...] = _dot(h, w2[...]) + b2[...]


def _node_enc_body(x_ref, w1, b1, w2, b2, para_ref, v_ref):
    h = jax.nn.silu(_dot(x_ref[...], w1[...]) + b1[...])
    v0 = _dot(h, w2[...]) + b2[...]
    gamma = para_ref[0:1, :ENC]
    beta = para_ref[0:1, ENC:]
    v_ref[...] = v0 * gamma + beta


def _edge_mlp_body(s_ref, r_ref, ef_ref, we1, be1, we2, be2, m_ref, efn_ref):
    ef = ef_ref[...]
    x = jnp.concatenate([s_ref[...], r_ref[...], ef], axis=-1)
    pre = _dot(x, we1[...]) + be1[...]
    m = _dot(jax.nn.silu(pre), we2[...]) + be2[...]
    m_ref[...] = m
    efn_ref[...] = ef + m


def _node_mlp_body(v_ref, a0_ref, a1_ref, wn1, bn1, wn2, bn2, vn_ref):
    v = v_ref[...]
    agg = a0_ref[0] + a1_ref[0]
    x = jnp.concatenate([v, agg], axis=-1)
    h = jax.nn.silu(_dot(x, wn1[...]) + bn1[...])
    vn_ref[...] = v + _dot(h, wn2[...]) + bn2[...]


def _decoder_body(x_ref, spread, v1_ref, v2_ref, v3_ref, v4_ref,
                  aw1, ab1, aw2, ab2, dw1, db1, dw2, db2, dw3, db3,
                  out_ref):
    x = x_ref[...]
    sp = spread[...]
    y = (x[:, 0:1] * sp[0:1, :] + x[:, 1:2] * sp[1:2, :]
         + x[:, 2:3] * sp[2:3, :])
    lane = lax.broadcasted_iota(jnp.int32, y.shape, 1)
    pe = jnp.where(lane < 18,
                   jnp.where(lane % 6 < 3, jnp.cos(y), jnp.sin(y)),
                   jnp.where(lane < 21, y, 0.0))
    a = _dot(jax.nn.silu(_dot(pe, aw1[...]) + ab1[...]), aw2[...]) + ab2[...]
    a = jnp.clip(a, -30.0, 30.0)
    valid = lane < 4
    al = jnp.where(valid, a, -1e30)
    mx = jnp.max(al, axis=1, keepdims=True)
    e = jnp.where(valid, jnp.exp(al - mx), 0.0)
    w = e / jnp.sum(e, axis=1, keepdims=True)
    vagg = (w[:, 0:1] * v1_ref[...] + w[:, 1:2] * v2_ref[...]
            + w[:, 2:3] * v3_ref[...] + w[:, 3:4] * v4_ref[...])
    dd = jnp.concatenate([vagg, pe], axis=-1)
    h = jax.nn.silu(_dot(dd, dw1[...]) + db1[...])
    h = jax.nn.silu(_dot(h, dw2[...]) + db2[...])
    o = _dot(h, dw3[...])
    res = o[:, 0:1] + db3[...][0:1, 0:1] + x[:, 3:4]
    out_ref[...] = jnp.broadcast_to(res, out_ref.shape)


def _full(shape):
    return pl.BlockSpec(shape, lambda i: (0,) * len(shape))


def _glob_call(t, c, sp, *ws):
    specs = [_full(x.shape) for x in (t, c, sp)] + [_full(w.shape) for w in ws]
    return pl.pallas_call(
        _glob_body,
        grid=(1,),
        in_specs=specs,
        out_specs=_full((8, 2 * ENC)),
        out_shape=jax.ShapeDtypeStruct((8, 2 * ENC), F32),
    )(t, c, sp, *ws)


def _edge_enc_call(spos, rpos, w1, b1, w2, b2):
    g = NE_PAD // ET
    espec = pl.BlockSpec((ET, 16), lambda i: (i, 0))
    return pl.pallas_call(
        _edge_enc_body,
        grid=(g,),
        in_specs=[espec, espec] + [_full(w.shape) for w in (w1, b1, w2, b2)],
        out_specs=pl.BlockSpec((ET, ENC), lambda i: (i, 0)),
        out_shape=jax.ShapeDtypeStruct((NE_PAD, ENC), F32),
    )(spos, rpos, w1, b1, w2, b2)


def _node_enc_call(x, w1, b1, w2, b2, para):
    g = N_NODES // NT
    nspec = pl.BlockSpec((NT, ENC), lambda i: (i, 0))
    return pl.pallas_call(
        _node_enc_body,
        grid=(g,),
        in_specs=[pl.BlockSpec((NT, 8), lambda i: (i, 0))]
        + [_full(w.shape) for w in (w1, b1, w2, b2, para)],
        out_specs=nspec,
        out_shape=jax.ShapeDtypeStruct((N_NODES, ENC), F32),
    )(x, w1, b1, w2, b2, para)


def _edge_mlp_call(s, r, ef, we1, be1, we2, be2):
    g = NE_PAD // ET
    espec = pl.BlockSpec((ET, ENC), lambda i: (i, 0))
    return pl.pallas_call(
        _edge_mlp_body,
        grid=(g,),
        in_specs=[espec, espec, espec]
        + [_full(w.shape) for w in (we1, be1, we2, be2)],
        out_specs=[espec, espec],
        out_shape=[jax.ShapeDtypeStruct((NE_PAD, ENC), F32)] * 2,
    )(s, r, ef, we1, be1, we2, be2)


def _node_mlp_call(v, aggs, wn1, bn1, wn2, bn2):
    g = N_NODES // NT
    nspec = pl.BlockSpec((NT, ENC), lambda i: (i, 0))
    a0spec = pl.BlockSpec((1, NT, ENC), lambda i: (0, i, 0))
    a1spec = pl.BlockSpec((1, NT, ENC), lambda i: (1, i, 0))
    return pl.pallas_call(
        _node_mlp_body,
        grid=(g,),
        in_specs=[nspec, a0spec, a1spec]
        + [_full(w.shape) for w in (wn1, bn1, wn2, bn2)],
        out_specs=nspec,
        out_shape=jax.ShapeDtypeStruct((N_NODES, ENC), F32),
    )(v, aggs, aggs, wn1, bn1, wn2, bn2)


def _decoder_call(x, spread, v1, v2, v3, v4, *ws):
    g = N_NODES // NT
    nspec = pl.BlockSpec((NT, ENC), lambda i: (i, 0))
    xspec = pl.BlockSpec((NT, 8), lambda i: (i, 0))
    return pl.pallas_call(
        _decoder_body,
        grid=(g,),
        in_specs=[xspec, _full(spread.shape), nspec, nspec, nspec, nspec]
        + [_full(w.shape) for w in ws],
        out_specs=xspec,
        out_shape=jax.ShapeDtypeStruct((N_NODES, 8), F32),
    )(x, spread, v1, v2, v3, v4, *ws)


def _row128(vec, width):
    out = jnp.zeros((8, 128), F32)
    return out.at[0, :width].set(vec)


def _padw(w, rows, cols):
    return jnp.zeros((rows, cols), F32).at[:w.shape[0], :w.shape[1]].set(w)


def kernel(node_pos, state_in, time_i, conditions, spatial_inform, edges, params):
    p = params
    pos = node_pos[0]
    state = state_in[0]

    send = edges[0, :, 0].astype(jnp.int32)
    recv = edges[0, :, 1].astype(jnp.int32)
    pad = NE_PAD - NE
    send_g = jnp.concatenate([send, jnp.zeros((pad,), jnp.int32)]).reshape(-1, CHUNK)
    recv_g = jnp.concatenate([recv, jnp.zeros((pad,), jnp.int32)]).reshape(-1, CHUNK)
    recv_s = jnp.concatenate(
        [recv, jnp.full((pad,), N_NODES, jnp.int32)]).reshape(-1, CHUNK)
    zeros_acc = jnp.zeros((R_PAD, ENC), F32)

    gather16 = _make_dual_gather(16)
    gather128 = _make_dual_gather(ENC)
    scatter_add = _make_scatter_add()

    para = _glob_call(
        _row128(time_i[0], 11), _row128(conditions[0], 32),
        _row128(spatial_inform[0], 10),
        _padw(p['ft_W1'], 128, 128), p['ft_b1'][None], p['ft_W2'], p['ft_b2'][None],
        _padw(p['fc_W1'], 128, 128), p['fc_b1'][None], p['fc_W2'], p['fc_b2'][None],
        _padw(p['fs_W1'], 128, 128), p['fs_b1'][None], p['fs_W2'], p['fs_b2'][None],
        p['fu_W1'], p['fu_b1'][None], p['fu_W2'], p['fu_b2'][None])

    pos16 = jnp.zeros((N_NODES, 16), F32).at[:, :S].set(pos)
    spos, rpos = gather16(pos16, send_g, pos16, recv_g)
    E = _edge_enc_call(
        spos, rpos, _padw(p['fe_W1'], 16, 128), p['fe_b1'][None],
        p['fe_W2'], p['fe_b2'][None])

    x_nodes = jnp.zeros((N_NODES, 8), F32).at[:, :S].set(pos)
    spread = jnp.zeros((8, 128), F32)
    freq = [float(jnp.pi), float(2 * jnp.pi), float(4 * jnp.pi)]
    for j in range(3):
        for k in range(3):
            spread = spread.at[j, 6 * j + k].set(freq[k])
            spread = spread.at[j, 6 * j + 3 + k].set(freq[k])
        spread = spread.at[j, 18 + j].set(1.0)

    outs = []
    for f in range(NF):
        xf = x_nodes.at[:, 3].set(state[:, f])
        v = _node_enc_call(
            xf, _padw(p[f'fvf{f}_W1'], 8, 128), p[f'fvf{f}_b1'][None],
            p[f'fvf{f}_W2'], p[f'fvf{f}_b2'][None], para)
        ef = E
        vs = []
        for b in range(NB):
            sg, rg = gather128(v, send_g, v, recv_g)
            m, ef = _edge_mlp_call(sg, rg, ef, p[f'blk{b}_We1'],
                                   p[f'blk{b}_be1'][None],
                                   p[f'blk{b}_We2'], p[f'blk{b}_be2'][None])
            aggs = scatter_add(m, recv_s, zeros_acc)
            v = _node_mlp_call(v, aggs, p[f'blk{b}_Wn1'], p[f'blk{b}_bn1'][None],
                               p[f'blk{b}_Wn2'], p[f'blk{b}_bn2'][None])
            vs.append(v)

        dw1 = jnp.concatenate(
            [p[f'dec{f}_dW1'][:ENC], _padw(p[f'dec{f}_dW1'][ENC:], 128, 128)],
            axis=0)
        out_f = _decoder_call(
            xf, spread, *vs,
            _padw(p[f'dec{f}_aW1'], 128, 128), p[f'dec{f}_ab1'][None],
            _padw(p[f'dec{f}_aW2'], 128, 128), _padw(p[f'dec{f}_ab2'][None], 1, 128),
            dw1, p[f'dec{f}_db1'][None], p[f'dec{f}_dW2'], p[f'dec{f}_db2'][None],
            _padw(p[f'dec{f}_dW3'], 128, 8), _padw(p[f'dec{f}_db3'][None], 1, 8))
        outs.append(out_f[:, 0])

    return jnp.stack(outs, axis=-1)[None]


# speedup vs baseline: 1.5894x; 1.0047x over previous
import functools

import jax
import jax.numpy as jnp
from jax import lax
from jax.experimental import pallas as pl
from jax.experimental.pallas import tpu as pltpu
from jax.experimental.pallas import tpu_sc as plsc

ENC = 128
NB = 4
NF = 2
S = 3
N_NODES = 10000
NE = 160000
NW = 32
CHUNK = 128
NE_PAD = 163840
K_CH = NE_PAD // NW // CHUNK
R_PAD = 10240
NT = 1000
ET = 1024

F32 = jnp.float32


def _make_dual_gather(cols):
    mesh = plsc.VectorSubcoreMesh(core_axis_name="c", subcore_axis_name="s")

    @functools.partial(
        pl.kernel,
        out_type=[jax.ShapeDtypeStruct((NE_PAD, cols), F32),
                  jax.ShapeDtypeStruct((NE_PAD, cols), F32)],
        mesh=mesh,
        compiler_params=pltpu.CompilerParams(use_tc_tiling_on_sc=(cols % 128 == 0)),
        scratch_types=[
            pltpu.VMEM((K_CH, CHUNK), jnp.int32),
            pltpu.VMEM((K_CH, CHUNK), jnp.int32),
            pltpu.VMEM((CHUNK, cols), F32),
            pltpu.VMEM((CHUNK, cols), F32),
            pltpu.VMEM((CHUNK, cols), F32),
            pltpu.VMEM((CHUNK, cols), F32),
            pltpu.SemaphoreType.DMA,
            pltpu.SemaphoreType.DMA,
            pltpu.SemaphoreType.DMA,
            pltpu.SemaphoreType.DMA,
            pltpu.SemaphoreType.DMA,
            pltpu.SemaphoreType.DMA,
            pltpu.SemaphoreType.DMA,
            pltpu.SemaphoreType.DMA,
        ],
    )
    def gather2(tab_a, idx_a, tab_b, idx_b, out_a, out_b,
                idx_va, idx_vb, buf_a0, buf_b0, buf_a1, buf_b1,
                sem_a0, sem_b0, sem_a1, sem_b1,
                wsem_a0, wsem_b0, wsem_a1, wsem_b1):
        cid = lax.axis_index("c")
        sid = lax.axis_index("s")
        wid = sid * 2 + cid
        base = wid * K_CH
        pltpu.sync_copy(idx_a.at[pl.ds(base, K_CH)], idx_va)
        pltpu.sync_copy(idx_b.at[pl.ds(base, K_CH)], idx_vb)

        def drain(buf, wsem):
            pltpu.make_async_copy(out_a.at[pl.ds(0, CHUNK)], buf, wsem).wait()

        @pl.loop(0, K_CH, step=2)
        def body(j):
            @pl.when(j > 0)
            def _():
                drain(buf_a0, wsem_a0)
                drain(buf_b0, wsem_b0)
                drain(buf_a1, wsem_a1)
                drain(buf_b1, wsem_b1)

            ca0 = pltpu.async_copy(tab_a.at[idx_va.at[j]], buf_a0, sem_a0)
            cb0 = pltpu.async_copy(tab_b.at[idx_vb.at[j]], buf_b0, sem_b0)
            ca1 = pltpu.async_copy(tab_a.at[idx_va.at[j + 1]], buf_a1, sem_a1)
            cb1 = pltpu.async_copy(tab_b.at[idx_vb.at[j + 1]], buf_b1, sem_b1)
            row0 = (base + j) * CHUNK
            ca0.wait()
            cb0.wait()
            pltpu.async_copy(buf_a0, out_a.at[pl.ds(row0, CHUNK)], wsem_a0)
            pltpu.async_copy(buf_b0, out_b.at[pl.ds(row0, CHUNK)], wsem_b0)
            ca1.wait()
            cb1.wait()
            pltpu.async_copy(buf_a1, out_a.at[pl.ds(row0 + CHUNK, CHUNK)], wsem_a1)
            pltpu.async_copy(buf_b1, out_b.at[pl.ds(row0 + CHUNK, CHUNK)], wsem_b1)

        drain(buf_a0, wsem_a0)
        drain(buf_b0, wsem_b0)
        drain(buf_a1, wsem_a1)
        drain(buf_b1, wsem_b1)

    return gather2


def _make_scatter_add():
    mesh = plsc.VectorSubcoreMesh(core_axis_name="c", subcore_axis_name="s")
    stripe = R_PAD // 16

    @functools.partial(
        pl.kernel,
        out_type=jax.ShapeDtypeStruct((2, R_PAD, ENC), F32),
        mesh=mesh,
        scratch_types=[
            pltpu.VMEM((K_CH, CHUNK), jnp.int32),
            pltpu.VMEM((CHUNK, ENC), F32),
            pltpu.VMEM((CHUNK, ENC), F32),
            pltpu.VMEM_SHARED((R_PAD, ENC), F32),
            pltpu.SemaphoreType.DMA,
            pltpu.SemaphoreType.DMA,
            pltpu.SemaphoreType.DMA,
            pltpu.SemaphoreType.DMA,
        ],
    )
    def scatter(m_hbm, idx_hbm, zeros_hbm, out, idx_v, buf0, buf1, shared,
                sem0, sem1, asem0, asem1):
        cid = lax.axis_index("c")
        sid = lax.axis_index("s")
        wid = sid * 2 + cid
        pltpu.sync_copy(zeros_hbm.at[pl.ds(sid * stripe, stripe)],
                        shared.at[pl.ds(sid * stripe, stripe)])
        plsc.subcore_barrier()
        base = wid * K_CH
        pltpu.sync_copy(idx_hbm.at[pl.ds(base, K_CH)], idx_v)

        def drain(buf, asem):
            pltpu.make_async_copy(m_hbm.at[pl.ds(0, CHUNK)], buf, asem).wait()

        @pl.loop(0, K_CH, step=2)
        def body(j):
            @pl.when(j > 0)
            def _():
                drain(buf0, asem0)
                drain(buf1, asem1)
            c0 = pltpu.async_copy(m_hbm.at[pl.ds((base + j) * CHUNK, CHUNK)],
                                  buf0, sem0)
            c1 = pltpu.async_copy(m_hbm.at[pl.ds((base + j + 1) * CHUNK, CHUNK)],
                                  buf1, sem1)
            c0.wait()
            pltpu.async_copy(buf0, shared.at[idx_v.at[j]], asem0, add=True)
            c1.wait()
            pltpu.async_copy(buf1, shared.at[idx_v.at[j + 1]], asem1, add=True)

        drain(buf0, asem0)
        drain(buf1, asem1)
        plsc.subcore_barrier()
        pltpu.sync_copy(shared.at[pl.ds(sid * stripe, stripe)],
                        out.at[cid].at[pl.ds(sid * stripe, stripe)])

    return scatter


def _dot(a, b):
    return jnp.dot(a, b, preferred_element_type=F32)


def _glob_body(t_ref, c_ref, sp_ref,
               wt1, bt1, wt2, bt2, wc1, bc1, wc2, bc2, ws1, bs1, ws2, bs2,
               fw1, fb1, fw2, fb2, para_ref):
    te = _dot(jax.nn.silu(_dot(t_ref[...], wt1[...]) + bt1[...]), wt2[...]) + bt2[...]
    ce = _dot(jax.nn.silu(_dot(c_ref[...], wc1[...]) + bc1[...]), wc2[...]) + bc2[...]
    se = _dot(jax.nn.silu(_dot(sp_ref[...], ws1[...]) + bs1[...]), ws2[...]) + bs2[...]
    h = jnp.concatenate([ce, te, se], axis=-1)
    hh = jax.nn.silu(_dot(h, fw1[...]) + fb1[...])
    para_ref[...] = _dot(hh, fw2[...]) + fb2[...]


def _edge_enc_body(spos_ref, rpos_ref, w1, b1, w2, b2, e_ref):
    d = rpos_ref[...] - spos_ref[...]
    nsq = jnp.sum(d * d, axis=1, keepdims=True) + 1e-8
    nrm = jnp.sqrt(nsq)
    d3 = d[:, :3]
    feat = jnp.concatenate(
        [d3, -d3, nrm, jnp.zeros((d.shape[0], 9), F32)], axis=-1)
    h = jax.nn.silu(_dot(feat, w1[...]) + b1[...])
    e_ref[...] = _dot(h, w2[...]) + b2[...]


def _node_enc_body(x_ref, w1, b1, w2, b2, para_ref, v_ref):
    h = jax.nn.silu(_dot(x_ref[...], w1[...]) + b1[...])
    v0 = _dot(h, w2[...]) + b2[...]
    gamma = para_ref[0:1, :ENC]
    beta = para_ref[0:1, ENC:]
    v_ref[...] = v0 * gamma + beta


def _edge_mlp_body(s_ref, r_ref, ef_ref, we1, be1, we2, be2, m_ref, efn_ref):
    ef = ef_ref[...]
    x = jnp.concatenate([s_ref[...], r_ref[...], ef], axis=-1)
    pre = _dot(x, we1[...]) + be1[...]
    m = _dot(jax.nn.silu(pre), we2[...]) + be2[...]
    m_ref[...] = m
    efn_ref[...] = ef + m


def _node_mlp_body(v_ref, a0_ref, a1_ref, wn1, bn1, wn2, bn2, vn_ref):
    v = v_ref[...]
    agg = a0_ref[0] + a1_ref[0]
    x = jnp.concatenate([v, agg], axis=-1)
    h = jax.nn.silu(_dot(x, wn1[...]) + bn1[...])
    vn_ref[...] = v + _dot(h, wn2[...]) + bn2[...]


def _decoder_body(x_ref, spread, v1_ref, v2_ref, v3_ref, v4_ref,
                  aw1, ab1, aw2, ab2, dw1, db1, dw2, db2, dw3, db3,
                  out_ref):
    x = x_ref[...]
    sp = spread[...]
    y = (x[:, 0:1] * sp[0:1, :] + x[:, 1:2] * sp[1:2, :]
         + x[:, 2:3] * sp[2:3, :])
    lane = lax.broadcasted_iota(jnp.int32, y.shape, 1)
    pe = jnp.where(lane < 18,
                   jnp.where(lane % 6 < 3, jnp.cos(y), jnp.sin(y)),
                   jnp.where(lane < 21, y, 0.0))
    a = _dot(jax.nn.silu(_dot(pe, aw1[...]) + ab1[...]), aw2[...]) + ab2[...]
    a = jnp.clip(a, -30.0, 30.0)
    valid = lane < 4
    al = jnp.where(valid, a, -1e30)
    mx = jnp.max(al, axis=1, keepdims=True)
    e = jnp.where(valid, jnp.exp(al - mx), 0.0)
    w = e / jnp.sum(e, axis=1, keepdims=True)
    vagg = (w[:, 0:1] * v1_ref[...] + w[:, 1:2] * v2_ref[...]
            + w[:, 2:3] * v3_ref[...] + w[:, 3:4] * v4_ref[...])
    dd = jnp.concatenate([vagg, pe], axis=-1)
    h = jax.nn.silu(_dot(dd, dw1[...]) + db1[...])
    h = jax.nn.silu(_dot(h, dw2[...]) + db2[...])
    o = _dot(h, dw3[...])
    res = o[:, 0:1] + db3[...][0:1, 0:1] + x[:, 3:4]
    out_ref[...] = jnp.broadcast_to(res, out_ref.shape)


def _full(shape):
    return pl.BlockSpec(shape, lambda i: (0,) * len(shape))


def _glob_call(t, c, sp, *ws):
    specs = [_full(x.shape) for x in (t, c, sp)] + [_full(w.shape) for w in ws]
    return pl.pallas_call(
        _glob_body,
        grid=(1,),
        in_specs=specs,
        out_specs=_full((8, 2 * ENC)),
        out_shape=jax.ShapeDtypeStruct((8, 2 * ENC), F32),
    )(t, c, sp, *ws)


def _edge_enc_call(spos, rpos, w1, b1, w2, b2):
    g = NE_PAD // ET
    espec = pl.BlockSpec((ET, 16), lambda i: (i, 0))
    return pl.pallas_call(
        _edge_enc_body,
        grid=(g,),
        in_specs=[espec, espec] + [_full(w.shape) for w in (w1, b1, w2, b2)],
        out_specs=pl.BlockSpec((ET, ENC), lambda i: (i, 0)),
        out_shape=jax.ShapeDtypeStruct((NE_PAD, ENC), F32),
    )(spos, rpos, w1, b1, w2, b2)


def _node_enc_call(x, w1, b1, w2, b2, para):
    g = N_NODES // NT
    nspec = pl.BlockSpec((NT, ENC), lambda i: (i, 0))
    return pl.pallas_call(
        _node_enc_body,
        grid=(g,),
        in_specs=[pl.BlockSpec((NT, 8), lambda i: (i, 0))]
        + [_full(w.shape) for w in (w1, b1, w2, b2, para)],
        out_specs=nspec,
        out_shape=jax.ShapeDtypeStruct((N_NODES, ENC), F32),
    )(x, w1, b1, w2, b2, para)


def _edge_mlp_call(s, r, ef, we1, be1, we2, be2):
    g = NE_PAD // ET
    espec = pl.BlockSpec((ET, ENC), lambda i: (i, 0))
    return pl.pallas_call(
        _edge_mlp_body,
        grid=(g,),
        in_specs=[espec, espec, espec]
        + [_full(w.shape) for w in (we1, be1, we2, be2)],
        out_specs=[espec, espec],
        out_shape=[jax.ShapeDtypeStruct((NE_PAD, ENC), F32)] * 2,
    )(s, r, ef, we1, be1, we2, be2)


def _node_mlp_call(v, aggs, wn1, bn1, wn2, bn2):
    g = N_NODES // NT
    nspec = pl.BlockSpec((NT, ENC), lambda i: (i, 0))
    a0spec = pl.BlockSpec((1, NT, ENC), lambda i: (0, i, 0))
    a1spec = pl.BlockSpec((1, NT, ENC), lambda i: (1, i, 0))
    return pl.pallas_call(
        _node_mlp_body,
        grid=(g,),
        in_specs=[nspec, a0spec, a1spec]
        + [_full(w.shape) for w in (wn1, bn1, wn2, bn2)],
        out_specs=nspec,
        out_shape=jax.ShapeDtypeStruct((N_NODES, ENC), F32),
    )(v, aggs, aggs, wn1, bn1, wn2, bn2)


def _decoder_call(x, spread, v1, v2, v3, v4, *ws):
    g = N_NODES // NT
    nspec = pl.BlockSpec((NT, ENC), lambda i: (i, 0))
    xspec = pl.BlockSpec((NT, 8), lambda i: (i, 0))
    return pl.pallas_call(
        _decoder_body,
        grid=(g,),
        in_specs=[xspec, _full(spread.shape), nspec, nspec, nspec, nspec]
        + [_full(w.shape) for w in ws],
        out_specs=xspec,
        out_shape=jax.ShapeDtypeStruct((N_NODES, 8), F32),
    )(x, spread, v1, v2, v3, v4, *ws)


def _row128(vec, width):
    out = jnp.zeros((8, 128), F32)
    return out.at[0, :width].set(vec)


def _padw(w, rows, cols):
    return jnp.zeros((rows, cols), F32).at[:w.shape[0], :w.shape[1]].set(w)


def kernel(node_pos, state_in, time_i, conditions, spatial_inform, edges, params):
    p = params
    pos = node_pos[0]
    state = state_in[0]

    send = edges[0, :, 0].astype(jnp.int32)
    recv = edges[0, :, 1].astype(jnp.int32)
    pad = NE_PAD - NE
    send_g = jnp.concatenate([send, jnp.zeros((pad,), jnp.int32)]).reshape(-1, CHUNK)
    recv_g = jnp.concatenate([recv, jnp.zeros((pad,), jnp.int32)]).reshape(-1, CHUNK)
    recv_s = jnp.concatenate(
        [recv, jnp.full((pad,), N_NODES, jnp.int32)]).reshape(-1, CHUNK)
    zeros_acc = jnp.zeros((R_PAD, ENC), F32)

    gather16 = _make_dual_gather(16)
    gather128 = _make_dual_gather(ENC)
    scatter_add = _make_scatter_add()

    para = _glob_call(
        _row128(time_i[0], 11), _row128(conditions[0], 32),
        _row128(spatial_inform[0], 10),
        _padw(p['ft_W1'], 128, 128), p['ft_b1'][None], p['ft_W2'], p['ft_b2'][None],
        _padw(p['fc_W1'], 128, 128), p['fc_b1'][None], p['fc_W2'], p['fc_b2'][None],
        _padw(p['fs_W1'], 128, 128), p['fs_b1'][None], p['fs_W2'], p['fs_b2'][None],
        p['fu_W1'], p['fu_b1'][None], p['fu_W2'], p['fu_b2'][None])

    pos16 = jnp.zeros((N_NODES, 16), F32).at[:, :S].set(pos)
    spos, rpos = gather16(pos16, send_g, pos16, recv_g)
    E = _edge_enc_call(
        spos, rpos, _padw(p['fe_W1'], 16, 128), p['fe_b1'][None],
        p['fe_W2'], p['fe_b2'][None])

    x_nodes = jnp.zeros((N_NODES, 8), F32).at[:, :S].set(pos)
    spread = jnp.zeros((8, 128), F32)
    freq = [float(jnp.pi), float(2 * jnp.pi), float(4 * jnp.pi)]
    for j in range(3):
        for k in range(3):
            spread = spread.at[j, 6 * j + k].set(freq[k])
            spread = spread.at[j, 6 * j + 3 + k].set(freq[k])
        spread = spread.at[j, 18 + j].set(1.0)

    outs = []
    for f in range(NF):
        xf = x_nodes.at[:, 3].set(state[:, f])
        v = _node_enc_call(
            xf, _padw(p[f'fvf{f}_W1'], 8, 128), p[f'fvf{f}_b1'][None],
            p[f'fvf{f}_W2'], p[f'fvf{f}_b2'][None], para)
        ef = E
        vs = []
        for b in range(NB):
            sg, rg = gather128(v, send_g, v, recv_g)
            m, ef = _edge_mlp_call(sg, rg, ef, p[f'blk{b}_We1'],
                                   p[f'blk{b}_be1'][None],
                                   p[f'blk{b}_We2'], p[f'blk{b}_be2'][None])
            aggs = scatter_add(m, recv_s, zeros_acc)
            v = _node_mlp_call(v, aggs, p[f'blk{b}_Wn1'], p[f'blk{b}_bn1'][None],
                               p[f'blk{b}_Wn2'], p[f'blk{b}_bn2'][None])
            vs.append(v)

        dw1 = jnp.concatenate(
            [p[f'dec{f}_dW1'][:ENC], _padw(p[f'dec{f}_dW1'][ENC:], 128, 128)],
            axis=0)
        out_f = _decoder_call(
            xf, spread, *vs,
            _padw(p[f'dec{f}_aW1'], 128, 128), p[f'dec{f}_ab1'][None],
            _padw(p[f'dec{f}_aW2'], 128, 128), _padw(p[f'dec{f}_ab2'][None], 1, 128),
            dw1, p[f'dec{f}_db1'][None], p[f'dec{f}_dW2'], p[f'dec{f}_db2'][None],
            _padw(p[f'dec{f}_dW3'], 128, 8), _padw(p[f'dec{f}_db3'][None], 1, 8))
        outs.append(out_f[:, 0])

    return jnp.stack(outs, axis=-1)[None]
